# baseline (speedup 1.0000x reference)
import functools

import jax
import jax.numpy as jnp
import numpy as np
from jax import lax
from jax.experimental import pallas as pl
from jax.experimental.pallas import tpu as pltpu
from jax.experimental.pallas import tpu_sc as plsc

N = 10000
E = 320000
D = 128
MAX_DEPTH = 32

NC = 2
NS = 16
L = 16
NP = 10240
ROWS_PT = NP // NS
CH = 80
EDGES_PT = E // (NC * NS)
NCHUNK = EDGES_PT // CH

BN = 400
GRID = N // BN
_MESH = plsc.VectorSubcoreMesh(
    core_axis_name="c", subcore_axis_name="s", num_cores=NC, num_subcores=NS)


EPT1 = E // NS


@functools.partial(
    pl.kernel,
    out_type=[
        jax.ShapeDtypeStruct((E,), jnp.float32),
        jax.ShapeDtypeStruct((NP,), jnp.float32),
    ],
    mesh=_MESH,
    scratch_types=(
        [pltpu.VMEM((NP,), jnp.float32)]
        + [pltpu.VMEM((NS, ROWS_PT), jnp.float32)]
        + [pltpu.VMEM((NP,), jnp.float32)]
        + [pltpu.VMEM((NP,), jnp.float32)]
        + [pltpu.VMEM((EDGES_PT,), jnp.int32)]
        + [pltpu.VMEM((EDGES_PT,), jnp.int32)]
        + [pltpu.VMEM((EDGES_PT,), jnp.float32)]
        + [pltpu.VMEM_SHARED((NS, NP), jnp.float32)]
    ),
    compiler_params=pltpu.CompilerParams(needs_layout_passes=False),
)
def _sc_degnorm(row_hbm, col_hbm, nm_hbm, rdeg_hbm,
                degp_v, degb_v, dinv_v, rdeg_v, row_v, col_v,
                nm_v, deg_sh):
    c = lax.axis_index("c")
    s = lax.axis_index("s")
    dbase = s * EPT1

    def zero_step(g, carry):
        degp_v[pl.ds(g * L, L)] = jnp.zeros((L,), jnp.float32)
        return carry

    lax.fori_loop(0, NP // L, zero_step, 0)
    pltpu.sync_copy(row_hbm.at[pl.ds(dbase, EPT1 // 2)], row_v)
    pltpu.sync_copy(row_hbm.at[pl.ds(dbase + EPT1 // 2, EPT1 // 2)], col_v)
    ones16 = jnp.full((L,), 1.0, jnp.float32)

    def count_step(src):
        def body(g, carry):
            plsc.addupdate_scatter(degp_v, [src[pl.ds(g * L, L)]], ones16)
            return carry
        lax.fori_loop(0, (EPT1 // 2) // L, body, 0)

    count_step(row_v)
    count_step(col_v)
    pltpu.sync_copy(degp_v, deg_sh.at[s])
    plsc.subcore_barrier()
    pltpu.sync_copy(deg_sh.at[:, pl.ds(s * ROWS_PT, ROWS_PT)], degb_v)

    def colsum_step(m, carry):
        acc = degb_v[0, pl.ds(m * L, L)]
        for r in range(1, NS):
            acc = acc + degb_v[r, pl.ds(m * L, L)]
        nm_v[pl.ds(m * L, L)] = acc
        return carry

    lax.fori_loop(0, ROWS_PT // L, colsum_step, 0)
    pltpu.sync_copy(nm_v.at[pl.ds(0, ROWS_PT)],
                    deg_sh.at[0, pl.ds(s * ROWS_PT, ROWS_PT)])
    plsc.subcore_barrier()

    pltpu.sync_copy(deg_sh.at[0], dinv_v)

    def newton_step(g, carry):
        d = dinv_v[pl.ds(g * L, L)] + 1.0
        i = plsc.bitcast(d, jnp.int32)
        y = plsc.bitcast(0x5F3759DF - (i >> 1), jnp.float32)
        y = y * (1.5 - 0.5 * d * y * y)
        y = y * (1.5 - 0.5 * d * y * y)
        y = y * (1.5 - 0.5 * d * y * y)
        dinv_v[pl.ds(g * L, L)] = y
        rdeg_v[pl.ds(g * L, L)] = y * y
        return carry

    lax.fori_loop(0, NP // L, newton_step, 0)

    @pl.when(c == 0)
    def _():
        pltpu.sync_copy(rdeg_v.at[pl.ds(s * ROWS_PT, ROWS_PT)],
                        rdeg_hbm.at[pl.ds(s * ROWS_PT, ROWS_PT)])

    base0 = (c * NS + s) * EDGES_PT
    pltpu.sync_copy(row_hbm.at[pl.ds(base0, EDGES_PT)], row_v)
    pltpu.sync_copy(col_hbm.at[pl.ds(base0, EDGES_PT)], col_v)

    def norm_step(g, carry):
        ridx = row_v[pl.ds(g * L, L)]
        cidx = col_v[pl.ds(g * L, L)]
        nm_v[pl.ds(g * L, L)] = (plsc.load_gather(dinv_v, [ridx]) *
                                 plsc.load_gather(dinv_v, [cidx]))
        return carry

    lax.fori_loop(0, EDGES_PT // L, norm_step, 0)
    pltpu.sync_copy(nm_v, nm_hbm.at[pl.ds(base0, EDGES_PT)])


_NSLOT = 4


@functools.partial(
    pl.kernel,
    out_type=jax.ShapeDtypeStruct((NC, N, D), jnp.float32),
    mesh=_MESH,
    scratch_types=(
        [pltpu.VMEM((3 * D,), jnp.float32)]
        + [pltpu.VMEM((CH,), jnp.int32) for _ in range(_NSLOT)]
        + [pltpu.VMEM((CH,), jnp.int32) for _ in range(_NSLOT)]
        + [pltpu.VMEM((2 * CH,), jnp.float32) for _ in range(_NSLOT)]
        + [pltpu.VMEM((CH,), jnp.float32) for _ in range(_NSLOT)]
        + [pltpu.VMEM((CH, D), jnp.float32) for _ in range(2)]
        + [pltpu.VMEM((CH, D), jnp.float32) for _ in range(2)]
        + [pltpu.SemaphoreType.DMA for _ in range(_NSLOT + 4)]
        + [pltpu.VMEM_SHARED((N, D), jnp.float32)]
    ),
    compiler_params=pltpu.CompilerParams(needs_layout_passes=False),
)
def _sc_conv(xx_hbm, row_hbm, col_hbm, ea_hbm, nm_hbm, wb_hbm, zeros_hbm,
             out_hbm, wb_v,
             rb0, rb1, rb2, rb3, cb0, cb1, cb2, cb3,
             eb0, eb1, eb2, eb3, nb0, nb1, nb2, nb3,
             xr0, xr1, ms0, ms1,
             i0, i1, i2, i3, g0, g1, s0, s1, agg_sh):
    rb = [rb0, rb1, rb2, rb3]
    cb = [cb0, cb1, cb2, cb3]
    eb = [eb0, eb1, eb2, eb3]
    nb = [nb0, nb1, nb2, nb3]
    xr = [xr0, xr1]
    ms = [ms0, ms1]
    gsem = [g0, g1]
    ssem = [s0, s1]
    isem = [i0, i1, i2, i3]

    c = lax.axis_index("c")
    s = lax.axis_index("s")
    tile = c * NS + s
    base0 = tile * EDGES_PT
    last = NS - 1

    @pl.when(s < last)
    def _():
        pltpu.sync_copy(zeros_hbm, agg_sh.at[pl.ds(s * ROWS_PT, ROWS_PT)])

    @pl.when(s == last)
    def _():
        pltpu.sync_copy(zeros_hbm.at[pl.ds(0, N - last * ROWS_PT)],
                        agg_sh.at[pl.ds(last * ROWS_PT, N - last * ROWS_PT)])

    pltpu.sync_copy(wb_hbm, wb_v)
    plsc.subcore_barrier()

    w0 = [wb_v[pl.ds(j * L, L)] for j in range(D // L)]
    w1 = [wb_v[pl.ds(D + j * L, L)] for j in range(D // L)]
    bb = [wb_v[pl.ds(2 * D + j * L, L)] for j in range(D // L)]

    def idx_load(k, q):
        b = base0 + k * CH
        pltpu.async_copy(row_hbm.at[pl.ds(b, CH)], rb[q], isem[q])
        pltpu.async_copy(col_hbm.at[pl.ds(b, CH)], cb[q], isem[q])
        pltpu.async_copy(ea_hbm.at[pl.ds(2 * b, 2 * CH)], eb[q], isem[q])
        pltpu.async_copy(nm_hbm.at[pl.ds(b, CH)], nb[q], isem[q])

    def idx_wait(q):
        pltpu.make_async_copy(row_hbm.at[pl.ds(0, CH)], rb[q], isem[q]).wait()
        pltpu.make_async_copy(row_hbm.at[pl.ds(0, CH)], cb[q], isem[q]).wait()
        pltpu.make_async_copy(ea_hbm.at[pl.ds(0, 2 * CH)], eb[q],
                              isem[q]).wait()
        pltpu.make_async_copy(nm_hbm.at[pl.ds(0, CH)], nb[q], isem[q]).wait()

    def gather(q, b):
        pltpu.async_copy(xx_hbm.at[rb[q]], xr[b], gsem[b])

    def buf_wait(buf, sem):
        pltpu.make_async_copy(xx_hbm.at[pl.ds(0, CH)], buf, sem).wait()

    def scatter(q, b):
        pltpu.async_copy(ms[b], agg_sh.at[cb[q]], ssem[b], add=True)

    def compute(q, b):
        src, dst = xr[b], ms[b]

        def group_step(g, gcarry):
            nrm16 = nb[q][pl.ds(g * L, L)]
            va = eb[q][pl.ds(2 * L * g, L)]
            vb = eb[q][pl.ds(2 * L * g + L, L)]
            for t in range(L):
                pair = va if t < L // 2 else vb
                ea0 = pair[(2 * t) % L]
                ea1 = pair[(2 * t + 1) % L]
                nm = nrm16[t]
                e = g * L + t
                for j in range(D // L):
                    v = src[e, pl.ds(j * L, L)]
                    m = jnp.maximum(v + ea0 * w0[j] + ea1 * w1[j] + bb[j],
                                    0.0) * nm
                    dst[e, pl.ds(j * L, L)] = m
            return gcarry

        lax.fori_loop(0, CH // L, group_step, 0)

    idx_load(0, 0)
    idx_load(1, 1)
    idx_wait(0)
    gather(0, 0)

    def chunk_body(k, q, b, first=False, pf_pred=None):
        idx_wait((q + 1) % _NSLOT)
        gather((q + 1) % _NSLOT, 1 - b)
        buf_wait(xr[b], gsem[b])
        if not first:
            buf_wait(ms[b], ssem[b])
        if pf_pred is None:
            idx_load(k + 2, (q + 2) % _NSLOT)
        else:
            @pl.when(pf_pred)
            def _():
                idx_load(k + 2, (q + 2) % _NSLOT)
        compute(q, b)
        scatter(q, b)

    chunk_body(0, 0, 0, first=True)
    chunk_body(1, 1, 1, first=True)
    chunk_body(2, 2, 0)
    chunk_body(3, 3, 1)

    @pl.loop(4, NCHUNK - 1, step=4)
    def _(k4):
        chunk_body(k4, 0, 0)
        chunk_body(k4 + 1, 1, 1)
        chunk_body(k4 + 2, 2, 0)
        chunk_body(k4 + 3, 3, 1, pf_pred=k4 + 5 < NCHUNK)

    buf_wait(xr[0], gsem[0])
    buf_wait(ms[0], ssem[0])
    compute(0, 0)
    scatter(0, 0)
    buf_wait(ms[0], ssem[0])
    buf_wait(ms[1], ssem[1])
    plsc.subcore_barrier()

    @pl.when(s < last)
    def _():
        pltpu.sync_copy(agg_sh.at[pl.ds(s * ROWS_PT, ROWS_PT)],
                        out_hbm.at[c, pl.ds(s * ROWS_PT, ROWS_PT)])

    @pl.when(s == last)
    def _():
        pltpu.sync_copy(
            agg_sh.at[pl.ds(last * ROWS_PT, N - last * ROWS_PT)],
            out_hbm.at[c, pl.ds(last * ROWS_PT, N - last * ROWS_PT)])


def _tc_pre_body(x_ref, dep_ref, rdeg_ref, Wn_ref, dt_ref, Wl_ref, bl_ref,
                 r0_ref, xx_ref, sf_ref):
    d = dep_ref[0, 0, :]
    oh = (d[:, None] == lax.broadcasted_iota(jnp.int32, (BN, MAX_DEPTH), 1)
          ).astype(jnp.float32)
    h0 = (jnp.dot(x_ref[...], Wn_ref[...], preferred_element_type=jnp.float32)
          + jnp.dot(oh, dt_ref[...], preferred_element_type=jnp.float32))
    xx = jnp.dot(h0, Wl_ref[...],
                 preferred_element_type=jnp.float32) + bl_ref[...]
    rdeg = rdeg_ref[0, 0, :]
    xx_ref[...] = xx
    sf_ref[...] = jnp.maximum(xx + r0_ref[...], 0.0) * rdeg[:, None]


def _tc_mid_body(agg_ref, sf_ref, Wl_ref, bl_ref, r1_ref, rdeg_ref,
                 xx_ref, sf1_ref):
    h1 = jnp.maximum(agg_ref[0] + agg_ref[1] + sf_ref[...], 0.0)
    xx = jnp.dot(h1, Wl_ref[...],
                 preferred_element_type=jnp.float32) + bl_ref[...]
    rdeg = rdeg_ref[0, 0, :]
    xx_ref[...] = xx
    sf1_ref[...] = jnp.maximum(xx + r1_ref[...], 0.0) * rdeg[:, None]


def _tc_fin_body(agg_ref, sf_ref, out_ref):
    out_ref[...] = agg_ref[0] + agg_ref[1] + sf_ref[...]


_full = lambda shape: pl.BlockSpec(shape, lambda i: tuple(0 for _ in shape))
_rowblk = pl.BlockSpec((BN, D), lambda i: (i, 0))

_tc_pre = pl.pallas_call(
    _tc_pre_body,
    grid=(GRID,),
    in_specs=[
        _rowblk,
        pl.BlockSpec((1, 1, BN), lambda i: (i, 0, 0)),
        pl.BlockSpec((1, 1, BN), lambda i: (i, 0, 0)),
        _full((D, D)), _full((MAX_DEPTH, D)), _full((D, D)),
        _full((1, D)), _full((1, D)),
    ],
    out_specs=[_rowblk, _rowblk],
    out_shape=[
        jax.ShapeDtypeStruct((N, D), jnp.float32),
        jax.ShapeDtypeStruct((N, D), jnp.float32),
    ],
)

_tc_mid = pl.pallas_call(
    _tc_mid_body,
    grid=(GRID,),
    in_specs=[
        pl.BlockSpec((NC, BN, D), lambda i: (0, i, 0)),
        _rowblk,
        _full((D, D)), _full((1, D)), _full((1, D)),
        pl.BlockSpec((1, 1, BN), lambda i: (i, 0, 0)),
    ],
    out_specs=[_rowblk, _rowblk],
    out_shape=[
        jax.ShapeDtypeStruct((N, D), jnp.float32),
        jax.ShapeDtypeStruct((N, D), jnp.float32),
    ],
)

_tc_fin = pl.pallas_call(
    _tc_fin_body,
    grid=(GRID,),
    in_specs=[
        pl.BlockSpec((NC, BN, D), lambda i: (0, i, 0)),
        _rowblk,
    ],
    out_specs=_rowblk,
    out_shape=jax.ShapeDtypeStruct((N, D), jnp.float32),
)


def kernel(x, edge_index, edge_attr, node_depth, W_node, depth_tab,
           W_lin0, b_lin0, root0, W_edge0, b_edge0,
           W_lin1, b_lin1, root1, W_edge1, b_edge1):
    row = edge_index[0]
    col = edge_index[1]
    zeros2 = jnp.zeros((ROWS_PT, D), jnp.float32)
    wb0 = jnp.concatenate([W_edge0[0], W_edge0[1], b_edge0])
    wb1 = jnp.concatenate([W_edge1[0], W_edge1[1], b_edge1])

    norm, rdeg_np = _sc_degnorm(row, col)
    rdeg3 = rdeg_np[:N].reshape(GRID, 1, BN)
    depth3 = node_depth.reshape(GRID, 1, BN)

    xx0, self0 = _tc_pre(x, depth3, rdeg3, W_node, depth_tab, W_lin0,
                         b_lin0.reshape(1, D), root0)
    ea_flat = edge_attr.reshape(2 * E)
    agg0 = _sc_conv(xx0, row, col, ea_flat, norm, wb0, zeros2)
    xx1, self1 = _tc_mid(agg0, self0, W_lin1, b_lin1.reshape(1, D),
                         root1, rdeg3)
    agg1 = _sc_conv(xx1, row, col, ea_flat, norm, wb1, zeros2)
    return _tc_fin(agg1, self1)


# speedup vs baseline: 1.0391x; 1.0391x over previous
import functools

import jax
import jax.numpy as jnp
import numpy as np
from jax import lax
from jax.experimental import pallas as pl
from jax.experimental.pallas import tpu as pltpu
from jax.experimental.pallas import tpu_sc as plsc

N = 10000
E = 320000
D = 128
MAX_DEPTH = 32

NC = 2
NS = 16
L = 16
NP = 10240
ROWS_PT = NP // NS
CH = 80
EDGES_PT = E // (NC * NS)
NCHUNK = EDGES_PT // CH

BN = 2000
GRID = N // BN
_MESH = plsc.VectorSubcoreMesh(
    core_axis_name="c", subcore_axis_name="s", num_cores=NC, num_subcores=NS)


EPT1 = E // NS


@functools.partial(
    pl.kernel,
    out_type=[
        jax.ShapeDtypeStruct((E,), jnp.float32),
        jax.ShapeDtypeStruct((NP,), jnp.float32),
    ],
    mesh=_MESH,
    scratch_types=(
        [pltpu.VMEM((NP,), jnp.float32)]
        + [pltpu.VMEM((NS, ROWS_PT), jnp.float32)]
        + [pltpu.VMEM((NP,), jnp.float32)]
        + [pltpu.VMEM((NP,), jnp.float32)]
        + [pltpu.VMEM((EDGES_PT,), jnp.int32)]
        + [pltpu.VMEM((EDGES_PT,), jnp.int32)]
        + [pltpu.VMEM((EDGES_PT,), jnp.float32)]
        + [pltpu.VMEM_SHARED((NS, NP), jnp.float32)]
    ),
    compiler_params=pltpu.CompilerParams(needs_layout_passes=False),
)
def _sc_degnorm(row_hbm, col_hbm, nm_hbm, rdeg_hbm,
                degp_v, degb_v, dinv_v, rdeg_v, row_v, col_v,
                nm_v, deg_sh):
    c = lax.axis_index("c")
    s = lax.axis_index("s")
    dbase = s * EPT1

    def zero_step(g, carry):
        degp_v[pl.ds(g * L, L)] = jnp.zeros((L,), jnp.float32)
        return carry

    lax.fori_loop(0, NP // L, zero_step, 0)
    pltpu.sync_copy(row_hbm.at[pl.ds(dbase, EPT1 // 2)], row_v)
    pltpu.sync_copy(row_hbm.at[pl.ds(dbase + EPT1 // 2, EPT1 // 2)], col_v)
    ones16 = jnp.full((L,), 1.0, jnp.float32)

    def count_step(src):
        def body(g, carry):
            plsc.addupdate_scatter(degp_v, [src[pl.ds(g * L, L)]], ones16)
            return carry
        lax.fori_loop(0, (EPT1 // 2) // L, body, 0)

    count_step(row_v)
    count_step(col_v)
    pltpu.sync_copy(degp_v, deg_sh.at[s])
    plsc.subcore_barrier()
    pltpu.sync_copy(deg_sh.at[:, pl.ds(s * ROWS_PT, ROWS_PT)], degb_v)

    def colsum_step(m, carry):
        acc = degb_v[0, pl.ds(m * L, L)]
        for r in range(1, NS):
            acc = acc + degb_v[r, pl.ds(m * L, L)]
        nm_v[pl.ds(m * L, L)] = acc
        return carry

    lax.fori_loop(0, ROWS_PT // L, colsum_step, 0)
    pltpu.sync_copy(nm_v.at[pl.ds(0, ROWS_PT)],
                    deg_sh.at[0, pl.ds(s * ROWS_PT, ROWS_PT)])
    plsc.subcore_barrier()

    pltpu.sync_copy(deg_sh.at[0], dinv_v)

    def newton_step(g, carry):
        d = dinv_v[pl.ds(g * L, L)] + 1.0
        i = plsc.bitcast(d, jnp.int32)
        y = plsc.bitcast(0x5F3759DF - (i >> 1), jnp.float32)
        y = y * (1.5 - 0.5 * d * y * y)
        y = y * (1.5 - 0.5 * d * y * y)
        y = y * (1.5 - 0.5 * d * y * y)
        dinv_v[pl.ds(g * L, L)] = y
        rdeg_v[pl.ds(g * L, L)] = y * y
        return carry

    lax.fori_loop(0, NP // L, newton_step, 0)

    @pl.when(c == 0)
    def _():
        pltpu.sync_copy(rdeg_v.at[pl.ds(s * ROWS_PT, ROWS_PT)],
                        rdeg_hbm.at[pl.ds(s * ROWS_PT, ROWS_PT)])

    base0 = (c * NS + s) * EDGES_PT
    pltpu.sync_copy(row_hbm.at[pl.ds(base0, EDGES_PT)], row_v)
    pltpu.sync_copy(col_hbm.at[pl.ds(base0, EDGES_PT)], col_v)

    def norm_step(g, carry):
        ridx = row_v[pl.ds(g * L, L)]
        cidx = col_v[pl.ds(g * L, L)]
        nm_v[pl.ds(g * L, L)] = (plsc.load_gather(dinv_v, [ridx]) *
                                 plsc.load_gather(dinv_v, [cidx]))
        return carry

    lax.fori_loop(0, EDGES_PT // L, norm_step, 0)
    pltpu.sync_copy(nm_v, nm_hbm.at[pl.ds(base0, EDGES_PT)])


_NSLOT = 4


@functools.partial(
    pl.kernel,
    out_type=jax.ShapeDtypeStruct((NC, N, D), jnp.float32),
    mesh=_MESH,
    scratch_types=(
        [pltpu.VMEM((3 * D,), jnp.float32)]
        + [pltpu.VMEM((CH,), jnp.int32) for _ in range(_NSLOT)]
        + [pltpu.VMEM((CH,), jnp.int32) for _ in range(_NSLOT)]
        + [pltpu.VMEM((2 * CH,), jnp.float32) for _ in range(_NSLOT)]
        + [pltpu.VMEM((CH,), jnp.float32) for _ in range(_NSLOT)]
        + [pltpu.VMEM((CH, D), jnp.float32) for _ in range(2)]
        + [pltpu.VMEM((CH, D), jnp.float32) for _ in range(2)]
        + [pltpu.SemaphoreType.DMA for _ in range(_NSLOT + 4)]
        + [pltpu.VMEM_SHARED((N, D), jnp.float32)]
    ),
    compiler_params=pltpu.CompilerParams(needs_layout_passes=False),
)
def _sc_conv(xx_hbm, row_hbm, col_hbm, ea_hbm, nm_hbm, wb_hbm, zeros_hbm,
             out_hbm, wb_v,
             rb0, rb1, rb2, rb3, cb0, cb1, cb2, cb3,
             eb0, eb1, eb2, eb3, nb0, nb1, nb2, nb3,
             xr0, xr1, ms0, ms1,
             i0, i1, i2, i3, g0, g1, s0, s1, agg_sh):
    rb = [rb0, rb1, rb2, rb3]
    cb = [cb0, cb1, cb2, cb3]
    eb = [eb0, eb1, eb2, eb3]
    nb = [nb0, nb1, nb2, nb3]
    xr = [xr0, xr1]
    ms = [ms0, ms1]
    gsem = [g0, g1]
    ssem = [s0, s1]
    isem = [i0, i1, i2, i3]

    c = lax.axis_index("c")
    s = lax.axis_index("s")
    tile = c * NS + s
    base0 = tile * EDGES_PT
    last = NS - 1

    @pl.when(s < last)
    def _():
        pltpu.sync_copy(zeros_hbm, agg_sh.at[pl.ds(s * ROWS_PT, ROWS_PT)])

    @pl.when(s == last)
    def _():
        pltpu.sync_copy(zeros_hbm.at[pl.ds(0, N - last * ROWS_PT)],
                        agg_sh.at[pl.ds(last * ROWS_PT, N - last * ROWS_PT)])

    pltpu.sync_copy(wb_hbm, wb_v)
    plsc.subcore_barrier()

    w0 = [wb_v[pl.ds(j * L, L)] for j in range(D // L)]
    w1 = [wb_v[pl.ds(D + j * L, L)] for j in range(D // L)]
    bb = [wb_v[pl.ds(2 * D + j * L, L)] for j in range(D // L)]

    def idx_load(k, q):
        b = base0 + k * CH
        pltpu.async_copy(row_hbm.at[pl.ds(b, CH)], rb[q], isem[q])
        pltpu.async_copy(col_hbm.at[pl.ds(b, CH)], cb[q], isem[q])
        pltpu.async_copy(ea_hbm.at[pl.ds(2 * b, 2 * CH)], eb[q], isem[q])
        pltpu.async_copy(nm_hbm.at[pl.ds(b, CH)], nb[q], isem[q])

    def idx_wait(q):
        pltpu.make_async_copy(row_hbm.at[pl.ds(0, CH)], rb[q], isem[q]).wait()
        pltpu.make_async_copy(row_hbm.at[pl.ds(0, CH)], cb[q], isem[q]).wait()
        pltpu.make_async_copy(ea_hbm.at[pl.ds(0, 2 * CH)], eb[q],
                              isem[q]).wait()
        pltpu.make_async_copy(nm_hbm.at[pl.ds(0, CH)], nb[q], isem[q]).wait()

    def gather(q, b):
        pltpu.async_copy(xx_hbm.at[rb[q]], xr[b], gsem[b])

    def buf_wait(buf, sem):
        pltpu.make_async_copy(xx_hbm.at[pl.ds(0, CH)], buf, sem).wait()

    def scatter(q, b):
        pltpu.async_copy(ms[b], agg_sh.at[cb[q]], ssem[b], add=True)

    def compute(q, b):
        src, dst = xr[b], ms[b]

        def group_step(g, gcarry):
            nrm16 = nb[q][pl.ds(g * L, L)]
            va = eb[q][pl.ds(2 * L * g, L)]
            vb = eb[q][pl.ds(2 * L * g + L, L)]
            for t in range(L):
                pair = va if t < L // 2 else vb
                ea0 = pair[(2 * t) % L]
                ea1 = pair[(2 * t + 1) % L]
                nm = nrm16[t]
                e = g * L + t
                for j in range(D // L):
                    v = src[e, pl.ds(j * L, L)]
                    m = jnp.maximum(v + ea0 * w0[j] + ea1 * w1[j] + bb[j],
                                    0.0) * nm
                    dst[e, pl.ds(j * L, L)] = m
            return gcarry

        lax.fori_loop(0, CH // L, group_step, 0)

    idx_load(0, 0)
    idx_load(1, 1)
    idx_wait(0)
    gather(0, 0)

    def chunk_body(k, q, b, first=False, pf_pred=None):
        idx_wait((q + 1) % _NSLOT)
        gather((q + 1) % _NSLOT, 1 - b)
        buf_wait(xr[b], gsem[b])
        if not first:
            buf_wait(ms[b], ssem[b])
        if pf_pred is None:
            idx_load(k + 2, (q + 2) % _NSLOT)
        else:
            @pl.when(pf_pred)
            def _():
                idx_load(k + 2, (q + 2) % _NSLOT)
        compute(q, b)
        scatter(q, b)

    chunk_body(0, 0, 0, first=True)
    chunk_body(1, 1, 1, first=True)
    chunk_body(2, 2, 0)
    chunk_body(3, 3, 1)

    @pl.loop(4, NCHUNK - 1, step=4)
    def _(k4):
        chunk_body(k4, 0, 0)
        chunk_body(k4 + 1, 1, 1)
        chunk_body(k4 + 2, 2, 0)
        chunk_body(k4 + 3, 3, 1, pf_pred=k4 + 5 < NCHUNK)

    buf_wait(xr[0], gsem[0])
    buf_wait(ms[0], ssem[0])
    compute(0, 0)
    scatter(0, 0)
    buf_wait(ms[0], ssem[0])
    buf_wait(ms[1], ssem[1])
    plsc.subcore_barrier()

    @pl.when(s < last)
    def _():
        pltpu.sync_copy(agg_sh.at[pl.ds(s * ROWS_PT, ROWS_PT)],
                        out_hbm.at[c, pl.ds(s * ROWS_PT, ROWS_PT)])

    @pl.when(s == last)
    def _():
        pltpu.sync_copy(
            agg_sh.at[pl.ds(last * ROWS_PT, N - last * ROWS_PT)],
            out_hbm.at[c, pl.ds(last * ROWS_PT, N - last * ROWS_PT)])


def _tc_pre_body(x_ref, dep_ref, rdeg_ref, Wn_ref, dt_ref, Wl_ref, bl_ref,
                 r0_ref, xx_ref, sf_ref):
    d = dep_ref[0, 0, :]
    oh = (d[:, None] == lax.broadcasted_iota(jnp.int32, (BN, MAX_DEPTH), 1)
          ).astype(jnp.float32)
    h0 = (jnp.dot(x_ref[...], Wn_ref[...], preferred_element_type=jnp.float32)
          + jnp.dot(oh, dt_ref[...], preferred_element_type=jnp.float32))
    xx = jnp.dot(h0, Wl_ref[...],
                 preferred_element_type=jnp.float32) + bl_ref[...]
    rdeg = rdeg_ref[0, 0, :]
    xx_ref[...] = xx
    sf_ref[...] = jnp.maximum(xx + r0_ref[...], 0.0) * rdeg[:, None]


def _tc_mid_body(agg_ref, sf_ref, Wl_ref, bl_ref, r1_ref, rdeg_ref,
                 xx_ref, sf1_ref):
    h1 = jnp.maximum(agg_ref[0] + agg_ref[1] + sf_ref[...], 0.0)
    xx = jnp.dot(h1, Wl_ref[...],
                 preferred_element_type=jnp.float32) + bl_ref[...]
    rdeg = rdeg_ref[0, 0, :]
    xx_ref[...] = xx
    sf1_ref[...] = jnp.maximum(xx + r1_ref[...], 0.0) * rdeg[:, None]


def _tc_fin_body(agg_ref, sf_ref, out_ref):
    out_ref[...] = agg_ref[0] + agg_ref[1] + sf_ref[...]


_full = lambda shape: pl.BlockSpec(shape, lambda i: tuple(0 for _ in shape))
_rowblk = pl.BlockSpec((BN, D), lambda i: (i, 0))

_tc_pre = pl.pallas_call(
    _tc_pre_body,
    grid=(GRID,),
    in_specs=[
        _rowblk,
        pl.BlockSpec((1, 1, BN), lambda i: (i, 0, 0)),
        pl.BlockSpec((1, 1, BN), lambda i: (i, 0, 0)),
        _full((D, D)), _full((MAX_DEPTH, D)), _full((D, D)),
        _full((1, D)), _full((1, D)),
    ],
    out_specs=[_rowblk, _rowblk],
    out_shape=[
        jax.ShapeDtypeStruct((N, D), jnp.float32),
        jax.ShapeDtypeStruct((N, D), jnp.float32),
    ],
)

_tc_mid = pl.pallas_call(
    _tc_mid_body,
    grid=(GRID,),
    in_specs=[
        pl.BlockSpec((NC, BN, D), lambda i: (0, i, 0)),
        _rowblk,
        _full((D, D)), _full((1, D)), _full((1, D)),
        pl.BlockSpec((1, 1, BN), lambda i: (i, 0, 0)),
    ],
    out_specs=[_rowblk, _rowblk],
    out_shape=[
        jax.ShapeDtypeStruct((N, D), jnp.float32),
        jax.ShapeDtypeStruct((N, D), jnp.float32),
    ],
)

_tc_fin = pl.pallas_call(
    _tc_fin_body,
    grid=(GRID,),
    in_specs=[
        pl.BlockSpec((NC, BN, D), lambda i: (0, i, 0)),
        _rowblk,
    ],
    out_specs=_rowblk,
    out_shape=jax.ShapeDtypeStruct((N, D), jnp.float32),
)


def kernel(x, edge_index, edge_attr, node_depth, W_node, depth_tab,
           W_lin0, b_lin0, root0, W_edge0, b_edge0,
           W_lin1, b_lin1, root1, W_edge1, b_edge1):
    row = edge_index[0]
    col = edge_index[1]
    zeros2 = jnp.zeros((ROWS_PT, D), jnp.float32)
    wb0 = jnp.concatenate([W_edge0[0], W_edge0[1], b_edge0])
    wb1 = jnp.concatenate([W_edge1[0], W_edge1[1], b_edge1])

    norm, rdeg_np = _sc_degnorm(row, col)
    rdeg3 = rdeg_np[:N].reshape(GRID, 1, BN)
    depth3 = node_depth.reshape(GRID, 1, BN)

    xx0, self0 = _tc_pre(x, depth3, rdeg3, W_node, depth_tab, W_lin0,
                         b_lin0.reshape(1, D), root0)
    ea_flat = edge_attr.reshape(2 * E)
    agg0 = _sc_conv(xx0, row, col, ea_flat, norm, wb0, zeros2)
    xx1, self1 = _tc_mid(agg0, self0, W_lin1, b_lin1.reshape(1, D),
                         root1, rdeg3)
    agg1 = _sc_conv(xx1, row, col, ea_flat, norm, wb1, zeros2)
    return _tc_fin(agg1, self1)


# speedup vs baseline: 1.0409x; 1.0017x over previous
import functools

import jax
import jax.numpy as jnp
import numpy as np
from jax import lax
from jax.experimental import pallas as pl
from jax.experimental.pallas import tpu as pltpu
from jax.experimental.pallas import tpu_sc as plsc

N = 10000
E = 320000
D = 128
MAX_DEPTH = 32

NC = 2
NS = 16
L = 16
NP = 10240
ROWS_PT = NP // NS
CH = 80
EDGES_PT = E // (NC * NS)
NCHUNK = EDGES_PT // CH

BN = 10000
GRID = N // BN
_MESH = plsc.VectorSubcoreMesh(
    core_axis_name="c", subcore_axis_name="s", num_cores=NC, num_subcores=NS)


EPT1 = E // NS


@functools.partial(
    pl.kernel,
    out_type=[
        jax.ShapeDtypeStruct((E,), jnp.float32),
        jax.ShapeDtypeStruct((NP,), jnp.float32),
    ],
    mesh=_MESH,
    scratch_types=(
        [pltpu.VMEM((NP,), jnp.float32)]
        + [pltpu.VMEM((NS, ROWS_PT), jnp.float32)]
        + [pltpu.VMEM((NP,), jnp.float32)]
        + [pltpu.VMEM((NP,), jnp.float32)]
        + [pltpu.VMEM((EDGES_PT,), jnp.int32)]
        + [pltpu.VMEM((EDGES_PT,), jnp.int32)]
        + [pltpu.VMEM((EDGES_PT,), jnp.float32)]
        + [pltpu.VMEM_SHARED((NS, NP), jnp.float32)]
    ),
    compiler_params=pltpu.CompilerParams(needs_layout_passes=False),
)
def _sc_degnorm(row_hbm, col_hbm, nm_hbm, rdeg_hbm,
                degp_v, degb_v, dinv_v, rdeg_v, row_v, col_v,
                nm_v, deg_sh):
    c = lax.axis_index("c")
    s = lax.axis_index("s")
    dbase = s * EPT1

    def zero_step(g, carry):
        degp_v[pl.ds(g * L, L)] = jnp.zeros((L,), jnp.float32)
        return carry

    lax.fori_loop(0, NP // L, zero_step, 0)
    pltpu.sync_copy(row_hbm.at[pl.ds(dbase, EPT1 // 2)], row_v)
    pltpu.sync_copy(row_hbm.at[pl.ds(dbase + EPT1 // 2, EPT1 // 2)], col_v)
    ones16 = jnp.full((L,), 1.0, jnp.float32)

    def count_step(src):
        def body(g, carry):
            plsc.addupdate_scatter(degp_v, [src[pl.ds(g * L, L)]], ones16)
            return carry
        lax.fori_loop(0, (EPT1 // 2) // L, body, 0)

    count_step(row_v)
    count_step(col_v)
    pltpu.sync_copy(degp_v, deg_sh.at[s])
    plsc.subcore_barrier()
    pltpu.sync_copy(deg_sh.at[:, pl.ds(s * ROWS_PT, ROWS_PT)], degb_v)

    def colsum_step(m, carry):
        acc = degb_v[0, pl.ds(m * L, L)]
        for r in range(1, NS):
            acc = acc + degb_v[r, pl.ds(m * L, L)]
        nm_v[pl.ds(m * L, L)] = acc
        return carry

    lax.fori_loop(0, ROWS_PT // L, colsum_step, 0)
    pltpu.sync_copy(nm_v.at[pl.ds(0, ROWS_PT)],
                    deg_sh.at[0, pl.ds(s * ROWS_PT, ROWS_PT)])
    plsc.subcore_barrier()

    pltpu.sync_copy(deg_sh.at[0], dinv_v)

    def newton_step(g, carry):
        d = dinv_v[pl.ds(g * L, L)] + 1.0
        i = plsc.bitcast(d, jnp.int32)
        y = plsc.bitcast(0x5F3759DF - (i >> 1), jnp.float32)
        y = y * (1.5 - 0.5 * d * y * y)
        y = y * (1.5 - 0.5 * d * y * y)
        y = y * (1.5 - 0.5 * d * y * y)
        dinv_v[pl.ds(g * L, L)] = y
        rdeg_v[pl.ds(g * L, L)] = y * y
        return carry

    lax.fori_loop(0, NP // L, newton_step, 0)

    @pl.when(c == 0)
    def _():
        pltpu.sync_copy(rdeg_v.at[pl.ds(s * ROWS_PT, ROWS_PT)],
                        rdeg_hbm.at[pl.ds(s * ROWS_PT, ROWS_PT)])

    base0 = (c * NS + s) * EDGES_PT
    pltpu.sync_copy(row_hbm.at[pl.ds(base0, EDGES_PT)], row_v)
    pltpu.sync_copy(col_hbm.at[pl.ds(base0, EDGES_PT)], col_v)

    def norm_step(g, carry):
        ridx = row_v[pl.ds(g * L, L)]
        cidx = col_v[pl.ds(g * L, L)]
        nm_v[pl.ds(g * L, L)] = (plsc.load_gather(dinv_v, [ridx]) *
                                 plsc.load_gather(dinv_v, [cidx]))
        return carry

    lax.fori_loop(0, EDGES_PT // L, norm_step, 0)
    pltpu.sync_copy(nm_v, nm_hbm.at[pl.ds(base0, EDGES_PT)])


_NSLOT = 4


@functools.partial(
    pl.kernel,
    out_type=jax.ShapeDtypeStruct((NC, N, D), jnp.float32),
    mesh=_MESH,
    scratch_types=(
        [pltpu.VMEM((3 * D,), jnp.float32)]
        + [pltpu.VMEM((CH,), jnp.int32) for _ in range(_NSLOT)]
        + [pltpu.VMEM((CH,), jnp.int32) for _ in range(_NSLOT)]
        + [pltpu.VMEM((2 * CH,), jnp.float32) for _ in range(_NSLOT)]
        + [pltpu.VMEM((CH,), jnp.float32) for _ in range(_NSLOT)]
        + [pltpu.VMEM((CH, D), jnp.float32) for _ in range(2)]
        + [pltpu.VMEM((CH, D), jnp.float32) for _ in range(2)]
        + [pltpu.SemaphoreType.DMA for _ in range(_NSLOT + 4)]
        + [pltpu.VMEM_SHARED((N, D), jnp.float32)]
    ),
    compiler_params=pltpu.CompilerParams(needs_layout_passes=False),
)
def _sc_conv(xx_hbm, row_hbm, col_hbm, ea_hbm, nm_hbm, wb_hbm, zeros_hbm,
             out_hbm, wb_v,
             rb0, rb1, rb2, rb3, cb0, cb1, cb2, cb3,
             eb0, eb1, eb2, eb3, nb0, nb1, nb2, nb3,
             xr0, xr1, ms0, ms1,
             i0, i1, i2, i3, g0, g1, s0, s1, agg_sh):
    rb = [rb0, rb1, rb2, rb3]
    cb = [cb0, cb1, cb2, cb3]
    eb = [eb0, eb1, eb2, eb3]
    nb = [nb0, nb1, nb2, nb3]
    xr = [xr0, xr1]
    ms = [ms0, ms1]
    gsem = [g0, g1]
    ssem = [s0, s1]
    isem = [i0, i1, i2, i3]

    c = lax.axis_index("c")
    s = lax.axis_index("s")
    tile = c * NS + s
    base0 = tile * EDGES_PT
    last = NS - 1

    @pl.when(s < last)
    def _():
        pltpu.sync_copy(zeros_hbm, agg_sh.at[pl.ds(s * ROWS_PT, ROWS_PT)])

    @pl.when(s == last)
    def _():
        pltpu.sync_copy(zeros_hbm.at[pl.ds(0, N - last * ROWS_PT)],
                        agg_sh.at[pl.ds(last * ROWS_PT, N - last * ROWS_PT)])

    pltpu.sync_copy(wb_hbm, wb_v)
    plsc.subcore_barrier()

    w0 = [wb_v[pl.ds(j * L, L)] for j in range(D // L)]
    w1 = [wb_v[pl.ds(D + j * L, L)] for j in range(D // L)]
    bb = [wb_v[pl.ds(2 * D + j * L, L)] for j in range(D // L)]

    def idx_load(k, q):
        b = base0 + k * CH
        pltpu.async_copy(row_hbm.at[pl.ds(b, CH)], rb[q], isem[q])
        pltpu.async_copy(col_hbm.at[pl.ds(b, CH)], cb[q], isem[q])
        pltpu.async_copy(ea_hbm.at[pl.ds(2 * b, 2 * CH)], eb[q], isem[q])
        pltpu.async_copy(nm_hbm.at[pl.ds(b, CH)], nb[q], isem[q])

    def idx_wait(q):
        pltpu.make_async_copy(row_hbm.at[pl.ds(0, CH)], rb[q], isem[q]).wait()
        pltpu.make_async_copy(row_hbm.at[pl.ds(0, CH)], cb[q], isem[q]).wait()
        pltpu.make_async_copy(ea_hbm.at[pl.ds(0, 2 * CH)], eb[q],
                              isem[q]).wait()
        pltpu.make_async_copy(nm_hbm.at[pl.ds(0, CH)], nb[q], isem[q]).wait()

    def gather(q, b):
        pltpu.async_copy(xx_hbm.at[rb[q]], xr[b], gsem[b])

    def buf_wait(buf, sem):
        pltpu.make_async_copy(xx_hbm.at[pl.ds(0, CH)], buf, sem).wait()

    def scatter(q, b):
        pltpu.async_copy(ms[b], agg_sh.at[cb[q]], ssem[b], add=True)

    def compute(q, b):
        src, dst = xr[b], ms[b]

        def group_step(g, gcarry):
            nrm16 = nb[q][pl.ds(g * L, L)]
            va = eb[q][pl.ds(2 * L * g, L)]
            vb = eb[q][pl.ds(2 * L * g + L, L)]
            for t in range(L):
                pair = va if t < L // 2 else vb
                ea0 = pair[(2 * t) % L]
                ea1 = pair[(2 * t + 1) % L]
                nm = nrm16[t]
                e = g * L + t
                for j in range(D // L):
                    v = src[e, pl.ds(j * L, L)]
                    m = jnp.maximum(v + ea0 * w0[j] + ea1 * w1[j] + bb[j],
                                    0.0) * nm
                    dst[e, pl.ds(j * L, L)] = m
            return gcarry

        lax.fori_loop(0, CH // L, group_step, 0)

    idx_load(0, 0)
    idx_load(1, 1)
    idx_wait(0)
    gather(0, 0)

    def chunk_body(k, q, b, first=False, pf_pred=None):
        idx_wait((q + 1) % _NSLOT)
        gather((q + 1) % _NSLOT, 1 - b)
        buf_wait(xr[b], gsem[b])
        if not first:
            buf_wait(ms[b], ssem[b])
        if pf_pred is None:
            idx_load(k + 2, (q + 2) % _NSLOT)
        else:
            @pl.when(pf_pred)
            def _():
                idx_load(k + 2, (q + 2) % _NSLOT)
        compute(q, b)
        scatter(q, b)

    chunk_body(0, 0, 0, first=True)
    chunk_body(1, 1, 1, first=True)
    chunk_body(2, 2, 0)
    chunk_body(3, 3, 1)

    @pl.loop(4, NCHUNK - 1, step=4)
    def _(k4):
        chunk_body(k4, 0, 0)
        chunk_body(k4 + 1, 1, 1)
        chunk_body(k4 + 2, 2, 0)
        chunk_body(k4 + 3, 3, 1, pf_pred=k4 + 5 < NCHUNK)

    buf_wait(xr[0], gsem[0])
    buf_wait(ms[0], ssem[0])
    compute(0, 0)
    scatter(0, 0)
    buf_wait(ms[0], ssem[0])
    buf_wait(ms[1], ssem[1])
    plsc.subcore_barrier()

    @pl.when(s < last)
    def _():
        pltpu.sync_copy(agg_sh.at[pl.ds(s * ROWS_PT, ROWS_PT)],
                        out_hbm.at[c, pl.ds(s * ROWS_PT, ROWS_PT)])

    @pl.when(s == last)
    def _():
        pltpu.sync_copy(
            agg_sh.at[pl.ds(last * ROWS_PT, N - last * ROWS_PT)],
            out_hbm.at[c, pl.ds(last * ROWS_PT, N - last * ROWS_PT)])


def _tc_pre_body(x_ref, dep_ref, rdeg_ref, Wn_ref, dt_ref, Wl_ref, bl_ref,
                 r0_ref, xx_ref, sf_ref):
    d = dep_ref[0, 0, :]
    oh = (d[:, None] == lax.broadcasted_iota(jnp.int32, (BN, MAX_DEPTH), 1)
          ).astype(jnp.float32)
    h0 = (jnp.dot(x_ref[...], Wn_ref[...], preferred_element_type=jnp.float32)
          + jnp.dot(oh, dt_ref[...], preferred_element_type=jnp.float32))
    xx = jnp.dot(h0, Wl_ref[...],
                 preferred_element_type=jnp.float32) + bl_ref[...]
    rdeg = rdeg_ref[0, 0, :]
    xx_ref[...] = xx
    sf_ref[...] = jnp.maximum(xx + r0_ref[...], 0.0) * rdeg[:, None]


def _tc_mid_body(agg_ref, sf_ref, Wl_ref, bl_ref, r1_ref, rdeg_ref,
                 xx_ref, sf1_ref):
    h1 = jnp.maximum(agg_ref[0] + agg_ref[1] + sf_ref[...], 0.0)
    xx = jnp.dot(h1, Wl_ref[...],
                 preferred_element_type=jnp.float32) + bl_ref[...]
    rdeg = rdeg_ref[0, 0, :]
    xx_ref[...] = xx
    sf1_ref[...] = jnp.maximum(xx + r1_ref[...], 0.0) * rdeg[:, None]


def _tc_fin_body(agg_ref, sf_ref, out_ref):
    out_ref[...] = agg_ref[0] + agg_ref[1] + sf_ref[...]


_full = lambda shape: pl.BlockSpec(shape, lambda i: tuple(0 for _ in shape))
_rowblk = pl.BlockSpec((BN, D), lambda i: (i, 0))

_tc_pre = pl.pallas_call(
    _tc_pre_body,
    grid=(GRID,),
    in_specs=[
        _rowblk,
        pl.BlockSpec((1, 1, BN), lambda i: (i, 0, 0)),
        pl.BlockSpec((1, 1, BN), lambda i: (i, 0, 0)),
        _full((D, D)), _full((MAX_DEPTH, D)), _full((D, D)),
        _full((1, D)), _full((1, D)),
    ],
    out_specs=[_rowblk, _rowblk],
    out_shape=[
        jax.ShapeDtypeStruct((N, D), jnp.float32),
        jax.ShapeDtypeStruct((N, D), jnp.float32),
    ],
)

_tc_mid = pl.pallas_call(
    _tc_mid_body,
    grid=(GRID,),
    in_specs=[
        pl.BlockSpec((NC, BN, D), lambda i: (0, i, 0)),
        _rowblk,
        _full((D, D)), _full((1, D)), _full((1, D)),
        pl.BlockSpec((1, 1, BN), lambda i: (i, 0, 0)),
    ],
    out_specs=[_rowblk, _rowblk],
    out_shape=[
        jax.ShapeDtypeStruct((N, D), jnp.float32),
        jax.ShapeDtypeStruct((N, D), jnp.float32),
    ],
)

_tc_fin = pl.pallas_call(
    _tc_fin_body,
    grid=(GRID,),
    in_specs=[
        pl.BlockSpec((NC, BN, D), lambda i: (0, i, 0)),
        _rowblk,
    ],
    out_specs=_rowblk,
    out_shape=jax.ShapeDtypeStruct((N, D), jnp.float32),
)


def kernel(x, edge_index, edge_attr, node_depth, W_node, depth_tab,
           W_lin0, b_lin0, root0, W_edge0, b_edge0,
           W_lin1, b_lin1, root1, W_edge1, b_edge1):
    row = edge_index[0]
    col = edge_index[1]
    zeros2 = jnp.zeros((ROWS_PT, D), jnp.float32)
    wb0 = jnp.concatenate([W_edge0[0], W_edge0[1], b_edge0])
    wb1 = jnp.concatenate([W_edge1[0], W_edge1[1], b_edge1])

    norm, rdeg_np = _sc_degnorm(row, col)
    rdeg3 = rdeg_np[:N].reshape(GRID, 1, BN)
    depth3 = node_depth.reshape(GRID, 1, BN)

    xx0, self0 = _tc_pre(x, depth3, rdeg3, W_node, depth_tab, W_lin0,
                         b_lin0.reshape(1, D), root0)
    ea_flat = edge_attr.reshape(2 * E)
    agg0 = _sc_conv(xx0, row, col, ea_flat, norm, wb0, zeros2)
    xx1, self1 = _tc_mid(agg0, self0, W_lin1, b_lin1.reshape(1, D),
                         root1, rdeg3)
    agg1 = _sc_conv(xx1, row, col, ea_flat, norm, wb1, zeros2)
    return _tc_fin(agg1, self1)


# speedup vs baseline: 1.0454x; 1.0043x over previous
import functools

import jax
import jax.numpy as jnp
import numpy as np
from jax import lax
from jax.experimental import pallas as pl
from jax.experimental.pallas import tpu as pltpu
from jax.experimental.pallas import tpu_sc as plsc

N = 10000
E = 320000
D = 128
MAX_DEPTH = 32

NC = 2
NS = 16
L = 16
NP = 10240
ROWS_PT = NP // NS
CH = 80
EDGES_PT = E // (NC * NS)
NCHUNK = EDGES_PT // CH

BN = 10000
GRID = N // BN
_MESH = plsc.VectorSubcoreMesh(
    core_axis_name="c", subcore_axis_name="s", num_cores=NC, num_subcores=NS)


EPT1 = E // NS


@functools.partial(
    pl.kernel,
    out_type=[
        jax.ShapeDtypeStruct((E,), jnp.float32),
        jax.ShapeDtypeStruct((NP,), jnp.float32),
    ],
    mesh=_MESH,
    scratch_types=(
        [pltpu.VMEM((NP,), jnp.float32)]
        + [pltpu.VMEM((NS, ROWS_PT), jnp.float32)]
        + [pltpu.VMEM((NP,), jnp.float32)]
        + [pltpu.VMEM((NP,), jnp.float32)]
        + [pltpu.VMEM((EDGES_PT,), jnp.int32)]
        + [pltpu.VMEM((EDGES_PT,), jnp.int32)]
        + [pltpu.VMEM((EDGES_PT,), jnp.float32)]
        + [pltpu.VMEM_SHARED((NS, NP), jnp.float32)]
    ),
    compiler_params=pltpu.CompilerParams(needs_layout_passes=False),
)
def _sc_degnorm(row_hbm, col_hbm, nm_hbm, rdeg_hbm,
                degp_v, degb_v, dinv_v, rdeg_v, row_v, col_v,
                nm_v, deg_sh):
    c = lax.axis_index("c")
    s = lax.axis_index("s")
    dbase = s * EPT1

    def zero_step(g, carry):
        degp_v[pl.ds(g * L, L)] = jnp.zeros((L,), jnp.float32)
        return carry

    lax.fori_loop(0, NP // L, zero_step, 0)
    pltpu.sync_copy(row_hbm.at[pl.ds(dbase, EPT1 // 2)], row_v)
    pltpu.sync_copy(row_hbm.at[pl.ds(dbase + EPT1 // 2, EPT1 // 2)], col_v)
    ones16 = jnp.full((L,), 1.0, jnp.float32)

    def count_step(src):
        def body(g, carry):
            plsc.addupdate_scatter(degp_v, [src[pl.ds(g * L, L)]], ones16)
            return carry
        lax.fori_loop(0, (EPT1 // 2) // L, body, 0)

    count_step(row_v)
    count_step(col_v)
    pltpu.sync_copy(degp_v, deg_sh.at[s])
    plsc.subcore_barrier()
    pltpu.sync_copy(deg_sh.at[:, pl.ds(s * ROWS_PT, ROWS_PT)], degb_v)

    def colsum_step(m, carry):
        acc = degb_v[0, pl.ds(m * L, L)]
        for r in range(1, NS):
            acc = acc + degb_v[r, pl.ds(m * L, L)]
        nm_v[pl.ds(m * L, L)] = acc
        return carry

    lax.fori_loop(0, ROWS_PT // L, colsum_step, 0)
    pltpu.sync_copy(nm_v.at[pl.ds(0, ROWS_PT)],
                    deg_sh.at[0, pl.ds(s * ROWS_PT, ROWS_PT)])
    plsc.subcore_barrier()

    pltpu.sync_copy(deg_sh.at[0], dinv_v)

    def newton_step(g, carry):
        d = dinv_v[pl.ds(g * L, L)] + 1.0
        i = plsc.bitcast(d, jnp.int32)
        y = plsc.bitcast(0x5F3759DF - (i >> 1), jnp.float32)
        y = y * (1.5 - 0.5 * d * y * y)
        y = y * (1.5 - 0.5 * d * y * y)
        y = y * (1.5 - 0.5 * d * y * y)
        dinv_v[pl.ds(g * L, L)] = y
        rdeg_v[pl.ds(g * L, L)] = y * y
        return carry

    lax.fori_loop(0, NP // L, newton_step, 0)

    @pl.when(c == 0)
    def _():
        pltpu.sync_copy(rdeg_v.at[pl.ds(s * ROWS_PT, ROWS_PT)],
                        rdeg_hbm.at[pl.ds(s * ROWS_PT, ROWS_PT)])

    base0 = (c * NS + s) * EDGES_PT
    pltpu.sync_copy(row_hbm.at[pl.ds(base0, EDGES_PT)], row_v)
    pltpu.sync_copy(col_hbm.at[pl.ds(base0, EDGES_PT)], col_v)

    def norm_step(g, carry):
        ridx = row_v[pl.ds(g * L, L)]
        cidx = col_v[pl.ds(g * L, L)]
        nm_v[pl.ds(g * L, L)] = (plsc.load_gather(dinv_v, [ridx]) *
                                 plsc.load_gather(dinv_v, [cidx]))
        return carry

    lax.fori_loop(0, EDGES_PT // L, norm_step, 0)
    pltpu.sync_copy(nm_v, nm_hbm.at[pl.ds(base0, EDGES_PT)])


_NSLOT = 4


@functools.partial(
    pl.kernel,
    out_type=jax.ShapeDtypeStruct((NC, N, D), jnp.float32),
    mesh=_MESH,
    scratch_types=(
        [pltpu.VMEM((3 * D,), jnp.float32)]
        + [pltpu.VMEM((CH,), jnp.int32) for _ in range(_NSLOT)]
        + [pltpu.VMEM((CH,), jnp.int32) for _ in range(_NSLOT)]
        + [pltpu.VMEM((2 * CH,), jnp.float32) for _ in range(_NSLOT)]
        + [pltpu.VMEM((CH,), jnp.float32) for _ in range(_NSLOT)]
        + [pltpu.VMEM((CH, D), jnp.float32) for _ in range(2)]
        + [pltpu.VMEM((CH, D), jnp.float32) for _ in range(2)]
        + [pltpu.SemaphoreType.DMA for _ in range(_NSLOT + 4)]
        + [pltpu.VMEM_SHARED((N, D), jnp.float32)]
    ),
    compiler_params=pltpu.CompilerParams(needs_layout_passes=False),
)
def _sc_conv(xx_hbm, row_hbm, col_hbm, ea_hbm, nm_hbm, wb_hbm, zeros_hbm,
             out_hbm, wb_v,
             rb0, rb1, rb2, rb3, cb0, cb1, cb2, cb3,
             eb0, eb1, eb2, eb3, nb0, nb1, nb2, nb3,
             xr0, xr1, ms0, ms1,
             i0, i1, i2, i3, g0, g1, s0, s1, agg_sh):
    rb = [rb0, rb1, rb2, rb3]
    cb = [cb0, cb1, cb2, cb3]
    eb = [eb0, eb1, eb2, eb3]
    nb = [nb0, nb1, nb2, nb3]
    xr = [xr0, xr1]
    ms = [ms0, ms1]
    gsem = [g0, g1]
    ssem = [s0, s1]
    isem = [i0, i1, i2, i3]

    c = lax.axis_index("c")
    s = lax.axis_index("s")
    tile = c * NS + s
    base0 = tile * EDGES_PT
    last = NS - 1

    @pl.when(s < last)
    def _():
        pltpu.sync_copy(zeros_hbm, agg_sh.at[pl.ds(s * ROWS_PT, ROWS_PT)])

    @pl.when(s == last)
    def _():
        pltpu.sync_copy(zeros_hbm.at[pl.ds(0, N - last * ROWS_PT)],
                        agg_sh.at[pl.ds(last * ROWS_PT, N - last * ROWS_PT)])

    pltpu.sync_copy(wb_hbm, wb_v)
    plsc.subcore_barrier()

    w0 = [wb_v[pl.ds(j * L, L)] for j in range(D // L)]
    w1 = [wb_v[pl.ds(D + j * L, L)] for j in range(D // L)]
    bb = [wb_v[pl.ds(2 * D + j * L, L)] for j in range(D // L)]

    def idx_load(k, q):
        b = base0 + k * CH
        pltpu.async_copy(row_hbm.at[pl.ds(b, CH)], rb[q], isem[q])
        pltpu.async_copy(col_hbm.at[pl.ds(b, CH)], cb[q], isem[q])
        pltpu.async_copy(ea_hbm.at[pl.ds(2 * b, 2 * CH)], eb[q], isem[q])
        pltpu.async_copy(nm_hbm.at[pl.ds(b, CH)], nb[q], isem[q])

    def idx_wait(q):
        pltpu.make_async_copy(row_hbm.at[pl.ds(0, CH)], rb[q], isem[q]).wait()
        pltpu.make_async_copy(row_hbm.at[pl.ds(0, CH)], cb[q], isem[q]).wait()
        pltpu.make_async_copy(ea_hbm.at[pl.ds(0, 2 * CH)], eb[q],
                              isem[q]).wait()
        pltpu.make_async_copy(nm_hbm.at[pl.ds(0, CH)], nb[q], isem[q]).wait()

    def gather(q, b):
        pltpu.async_copy(xx_hbm.at[rb[q]], xr[b], gsem[b])

    def buf_wait(buf, sem):
        pltpu.make_async_copy(xx_hbm.at[pl.ds(0, CH)], buf, sem).wait()

    def scatter(q, b):
        pltpu.async_copy(ms[b], agg_sh.at[cb[q]], ssem[b], add=True)

    def compute(q, b):
        src, dst = xr[b], ms[b]

        def group_step(g, gcarry):
            nrm16 = nb[q][pl.ds(g * L, L)]
            va = eb[q][pl.ds(2 * L * g, L)]
            vb = eb[q][pl.ds(2 * L * g + L, L)]
            for t in range(L):
                pair = va if t < L // 2 else vb
                ea0 = pair[(2 * t) % L]
                ea1 = pair[(2 * t + 1) % L]
                nm = nrm16[t]
                e = g * L + t
                for j in range(D // L):
                    v = src[e, pl.ds(j * L, L)]
                    m = jnp.maximum(v + ea0 * w0[j] + ea1 * w1[j] + bb[j],
                                    0.0) * nm
                    dst[e, pl.ds(j * L, L)] = m
            return gcarry

        lax.fori_loop(0, CH // L, group_step, 0)

    idx_load(0, 0)
    idx_load(1, 1)
    idx_wait(0)
    gather(0, 0)

    def chunk_body(k, q, b, first=False, pf_pred=None):
        idx_wait((q + 1) % _NSLOT)
        gather((q + 1) % _NSLOT, 1 - b)
        buf_wait(xr[b], gsem[b])
        if not first:
            buf_wait(ms[b], ssem[b])
        if pf_pred is None:
            idx_load(k + 2, (q + 2) % _NSLOT)
        else:
            @pl.when(pf_pred)
            def _():
                idx_load(k + 2, (q + 2) % _NSLOT)
        compute(q, b)
        scatter(q, b)

    chunk_body(0, 0, 0, first=True)
    chunk_body(1, 1, 1, first=True)
    chunk_body(2, 2, 0)
    chunk_body(3, 3, 1)

    @pl.loop(4, NCHUNK - 1, step=4)
    def _(k4):
        chunk_body(k4, 0, 0)
        chunk_body(k4 + 1, 1, 1)
        chunk_body(k4 + 2, 2, 0)
        chunk_body(k4 + 3, 3, 1, pf_pred=k4 + 5 < NCHUNK)

    buf_wait(xr[0], gsem[0])
    buf_wait(ms[0], ssem[0])
    compute(0, 0)
    scatter(0, 0)
    buf_wait(ms[0], ssem[0])
    buf_wait(ms[1], ssem[1])
    plsc.subcore_barrier()

    @pl.when(s < last)
    def _():
        pltpu.sync_copy(agg_sh.at[pl.ds(s * ROWS_PT, ROWS_PT)],
                        out_hbm.at[c, pl.ds(s * ROWS_PT, ROWS_PT)])

    @pl.when(s == last)
    def _():
        pltpu.sync_copy(
            agg_sh.at[pl.ds(last * ROWS_PT, N - last * ROWS_PT)],
            out_hbm.at[c, pl.ds(last * ROWS_PT, N - last * ROWS_PT)])


def _tc_pre_body(x_ref, dep_ref, Wn_ref, dt_ref, Wl_ref, bl_ref,
                 r0_ref, xx_ref, sf_ref):
    d = dep_ref[0, 0, :]
    oh = (d[:, None] == lax.broadcasted_iota(jnp.int32, (BN, MAX_DEPTH), 1)
          ).astype(jnp.float32)
    h0 = (jnp.dot(x_ref[...], Wn_ref[...], preferred_element_type=jnp.float32)
          + jnp.dot(oh, dt_ref[...], preferred_element_type=jnp.float32))
    xx = jnp.dot(h0, Wl_ref[...],
                 preferred_element_type=jnp.float32) + bl_ref[...]
    xx_ref[...] = xx
    sf_ref[...] = jnp.maximum(xx + r0_ref[...], 0.0)


def _tc_mid_body(agg_ref, sf_ref, Wl_ref, bl_ref, r1_ref, rdeg_ref,
                 xx_ref, sf1_ref):
    rdeg = rdeg_ref[0, 0, :]
    h1 = jnp.maximum(agg_ref[0] + agg_ref[1] + sf_ref[...] * rdeg[:, None],
                     0.0)
    xx = jnp.dot(h1, Wl_ref[...],
                 preferred_element_type=jnp.float32) + bl_ref[...]
    xx_ref[...] = xx
    sf1_ref[...] = jnp.maximum(xx + r1_ref[...], 0.0) * rdeg[:, None]


def _tc_fin_body(agg_ref, sf_ref, out_ref):
    out_ref[...] = agg_ref[0] + agg_ref[1] + sf_ref[...]


_full = lambda shape: pl.BlockSpec(shape, lambda i: tuple(0 for _ in shape))
_rowblk = pl.BlockSpec((BN, D), lambda i: (i, 0))

_tc_pre = pl.pallas_call(
    _tc_pre_body,
    grid=(GRID,),
    in_specs=[
        _rowblk,
        pl.BlockSpec((1, 1, BN), lambda i: (i, 0, 0)),
        _full((D, D)), _full((MAX_DEPTH, D)), _full((D, D)),
        _full((1, D)), _full((1, D)),
    ],
    out_specs=[_rowblk, _rowblk],
    out_shape=[
        jax.ShapeDtypeStruct((N, D), jnp.float32),
        jax.ShapeDtypeStruct((N, D), jnp.float32),
    ],
)

_tc_mid = pl.pallas_call(
    _tc_mid_body,
    grid=(GRID,),
    in_specs=[
        pl.BlockSpec((NC, BN, D), lambda i: (0, i, 0)),
        _rowblk,
        _full((D, D)), _full((1, D)), _full((1, D)),
        pl.BlockSpec((1, 1, BN), lambda i: (i, 0, 0)),
    ],
    out_specs=[_rowblk, _rowblk],
    out_shape=[
        jax.ShapeDtypeStruct((N, D), jnp.float32),
        jax.ShapeDtypeStruct((N, D), jnp.float32),
    ],
)

_tc_fin = pl.pallas_call(
    _tc_fin_body,
    grid=(GRID,),
    in_specs=[
        pl.BlockSpec((NC, BN, D), lambda i: (0, i, 0)),
        _rowblk,
    ],
    out_specs=_rowblk,
    out_shape=jax.ShapeDtypeStruct((N, D), jnp.float32),
)


def kernel(x, edge_index, edge_attr, node_depth, W_node, depth_tab,
           W_lin0, b_lin0, root0, W_edge0, b_edge0,
           W_lin1, b_lin1, root1, W_edge1, b_edge1):
    row = edge_index[0]
    col = edge_index[1]
    zeros2 = jnp.zeros((ROWS_PT, D), jnp.float32)
    wb0 = jnp.concatenate([W_edge0[0], W_edge0[1], b_edge0])
    wb1 = jnp.concatenate([W_edge1[0], W_edge1[1], b_edge1])

    norm, rdeg_np = _sc_degnorm(row, col)
    rdeg3 = rdeg_np[:N].reshape(GRID, 1, BN)
    depth3 = node_depth.reshape(GRID, 1, BN)

    xx0, self0 = _tc_pre(x, depth3, W_node, depth_tab, W_lin0,
                         b_lin0.reshape(1, D), root0)
    ea_flat = edge_attr.reshape(2 * E)
    agg0 = _sc_conv(xx0, row, col, ea_flat, norm, wb0, zeros2)
    xx1, self1 = _tc_mid(agg0, self0, W_lin1, b_lin1.reshape(1, D),
                         root1, rdeg3)
    agg1 = _sc_conv(xx1, row, col, ea_flat, norm, wb1, zeros2)
    return _tc_fin(agg1, self1)
